# baseline (speedup 1.0000x reference)
import functools
import math

import jax
import jax.numpy as jnp
from jax import lax
from jax.experimental import pallas as pl
from jax.experimental.pallas import tpu as pltpu
from jax.experimental.pallas import tpu_sc as plsc

USER_NUM = 60000
ITEM_NUM = 40000
N = USER_NUM + ITEM_NUM
E = 1600000
D = 32

NC = 2
NS = 16
LANES = 16

HALF = N // NC
RPT = HALF // NS
DUMMY = HALF
ACC_ROWS = HALF + 8

CH = 128
EBLK = 1536
CPB = EBLK // CH
NBLK = 68
EPT = NBLK * EBLK
E_PAD = EPT * NS
NBUF = 4

RCH = 125
NRCH = RPT // RCH

_ALPHA = 3.0 * math.tanh(1.0 / 3.0)
W1 = 2.0 * _ALPHA
W2 = 1.875 * _ALPHA
C2 = -0.75 * _ALPHA * _ALPHA
W3 = (28.0 / 15.0) * _ALPHA
C3 = -0.8 * _ALPHA * _ALPHA
BETA = 2.0

_MESH = plsc.VectorSubcoreMesh(core_axis_name="c", subcore_axis_name="s")
_CPARAMS = pltpu.CompilerParams(use_tc_tiling_on_sc=False,
                                needs_layout_passes=False)

_EDGE_SCRATCH = [
    pltpu.VMEM_SHARED((ACC_ROWS, D), jnp.float32),
    pltpu.VMEM((3 * EBLK,), jnp.int32),
    pltpu.VMEM((NBUF, CH), jnp.int32),
    pltpu.VMEM((NBUF, CH), jnp.int32),
    pltpu.VMEM((NBUF, CH + LANES), jnp.float32),
    pltpu.VMEM((NBUF, CH, D), jnp.float32),
    pltpu.VMEM((RCH, D), jnp.float32),
    pltpu.VMEM((RCH, D), jnp.float32),
] + [pltpu.SemaphoreType.DMA] * NBUF


def _zero_acc(s, acc, zbuf):
    z16 = jnp.zeros((LANES,), jnp.float32)

    def fill(r, carry):
        zbuf[r, pl.ds(0, 16)] = z16
        zbuf[r, pl.ds(16, 16)] = z16
        return carry

    lax.fori_loop(0, RCH, fill, 0)
    base = s * RPT

    def body(k, carry):
        pltpu.sync_copy(zbuf, acc.at[pl.ds(base + k * RCH, RCH)])
        return carry

    lax.fori_loop(0, NRCH, body, 0)
    @pl.when(s == 0)
    def _():
        pltpu.sync_copy(zbuf.at[pl.ds(0, 8)], acc.at[pl.ds(HALF, 8)])


def _edge_phase(c, s, pk_hbm, x_hbm, acc,
                eb, sidx, lidx, wbuf, rows, gsems, wscale):
    half_lo = c * HALF

    def _prep_and_issue(ki, b):
        o = pl.multiple_of(ki * CH, 8)

        def prep(g2, cc):
            for j in range(2):
                go = (g2 * 2 + j) * LANES
                sl = pl.ds(o + go, LANES)
                sidx[b, pl.ds(go, LANES)] = eb[sl]
                dv = eb[pl.ds(EBLK + o + go, LANES)] - half_lo
                ok = (dv >= 0) & (dv < HALF)
                lidx[b, pl.ds(go, LANES)] = jnp.where(ok, dv, DUMMY)
                wv = plsc.bitcast(eb[pl.ds(2 * EBLK + o + go, LANES)],
                                  jnp.float32)
                wbuf[b, pl.ds(go, LANES)] = wv * wscale
            return cc

        lax.fori_loop(0, CH // LANES // 2, prep, 0)
        pltpu.async_copy(x_hbm.at[sidx.at[b]], rows.at[b], gsems[b])

    def _finish(b):
        pltpu.make_async_copy(x_hbm.at[pl.ds(0, CH)], rows.at[b],
                              gsems[b]).wait()

        def scale(q, cc):
            for j in range(4):
                e = q * 4 + j
                ws = wbuf[b, pl.ds(e, LANES)][0]
                rows[b, e, pl.ds(0, 16)] = rows[b, e, pl.ds(0, 16)] * ws
                rows[b, e, pl.ds(16, 16)] = rows[b, e, pl.ds(16, 16)] * ws
            return cc

        lax.fori_loop(0, CH // 4, scale, 0)
        pltpu.sync_copy(rows.at[b], acc.at[lidx.at[b]], add=True)

    def block(bi, carry):
        pltpu.sync_copy(pk_hbm.at[s * NBLK + bi], eb)
        for b in range(NBUF - 1):
            _prep_and_issue(b, b)

        def ring(q, cc):
            for j in range(NBUF):
                ki = q * NBUF + j
                _finish(j)
                nxt = ki + NBUF - 1

                @pl.when(nxt < CPB)
                def _():
                    _prep_and_issue(nxt, (j + NBUF - 1) % NBUF)
            return cc

        lax.fori_loop(0, CPB // NBUF, ring, 0)
        return carry

    lax.fori_loop(0, NBLK, block, 0)


def _layer_kernel(wscale, cprev):
    @functools.partial(
        pl.kernel,
        mesh=_MESH,
        out_type=jax.ShapeDtypeStruct((N, D), jnp.float32),
        scratch_types=_EDGE_SCRATCH,
        compiler_params=_CPARAMS,
    )
    def k(*refs):
        if cprev is None:
            pk_hbm, xc_hbm, out_hbm = refs[:3]
            xp_hbm = None
            rest = refs[3:]
        else:
            pk_hbm, xc_hbm, xp_hbm, out_hbm = refs[:4]
            rest = refs[4:]
        acc, eb, sidx, lidx, wbuf, rows, ta, tb = rest[:8]
        gsems = rest[8:]

        c = lax.axis_index("c")
        s = lax.axis_index("s")

        _zero_acc(s, acc, ta)
        plsc.subcore_barrier()
        _edge_phase(c, s, pk_hbm, xc_hbm, acc,
                    eb, sidx, lidx, wbuf, rows, gsems, wscale)
        plsc.subcore_barrier()

        lbase = s * RPT
        gbase = c * HALF + lbase

        def wb(k_, carry):
            lrow = lbase + k_ * RCH
            grow = gbase + k_ * RCH
            pltpu.sync_copy(acc.at[pl.ds(lrow, RCH)], ta)
            if cprev is not None:
                pltpu.sync_copy(xp_hbm.at[pl.ds(grow, RCH)], tb)

                def axpy(r, cc):
                    for h in (0, 16):
                        sl = pl.ds(h, 16)
                        ta[r, sl] = ta[r, sl] + tb[r, sl] * cprev
                    return cc

                lax.fori_loop(0, RCH, axpy, 0)
            pltpu.sync_copy(ta, out_hbm.at[pl.ds(grow, RCH)])
            return carry

        lax.fori_loop(0, NRCH, wb, 0)

    return k


def _final_kernel(wscale, cprev):
    @functools.partial(
        pl.kernel,
        mesh=_MESH,
        out_type=(jax.ShapeDtypeStruct((N, D), jnp.float32),
                  jax.ShapeDtypeStruct((N, D), jnp.float32)),
        scratch_types=_EDGE_SCRATCH,
        compiler_params=_CPARAMS,
    )
    def k(pk_hbm, x0_hbm, x1_hbm, x2_hbm, low_hbm, mid_hbm,
          acc, eb, sidx, lidx, wbuf, rows, ta, tb, *gsems):
        c = lax.axis_index("c")
        s = lax.axis_index("s")

        _zero_acc(s, acc, ta)
        plsc.subcore_barrier()
        _edge_phase(c, s, pk_hbm, x2_hbm, acc,
                    eb, sidx, lidx, wbuf, rows, gsems, wscale)
        plsc.subcore_barrier()

        lbase = s * RPT
        gbase = c * HALF + lbase
        c1 = 1.0 + cprev

        def wb(k_, carry):
            lrow = lbase + k_ * RCH
            grow = gbase + k_ * RCH
            pltpu.sync_copy(acc.at[pl.ds(lrow, RCH)], ta)
            pltpu.sync_copy(x1_hbm.at[pl.ds(grow, RCH)], tb)

            def add1(r, cc):
                for h in (0, 16):
                    sl = pl.ds(h, 16)
                    ta[r, sl] = ta[r, sl] + tb[r, sl] * c1
                return cc

            lax.fori_loop(0, RCH, add1, 0)
            pltpu.sync_copy(x2_hbm.at[pl.ds(grow, RCH)], tb)

            def add2(r, cc):
                for h in (0, 16):
                    sl = pl.ds(h, 16)
                    ta[r, sl] = ta[r, sl] + tb[r, sl]
                return cc

            lax.fori_loop(0, RCH, add2, 0)
            pltpu.sync_copy(x0_hbm.at[pl.ds(grow, RCH)], tb)

            def final(r, cc):
                for h in (0, 16):
                    sl = pl.ds(h, 16)
                    low = (ta[r, sl] + tb[r, sl]) * 0.25
                    ta[r, sl] = low
                    tb[r, sl] = tb[r, sl] * BETA - low
                return cc

            lax.fori_loop(0, RCH, final, 0)
            pltpu.sync_copy(ta, low_hbm.at[pl.ds(grow, RCH)])
            pltpu.sync_copy(tb, mid_hbm.at[pl.ds(grow, RCH)])
            return carry

        lax.fori_loop(0, NRCH, wb, 0)

    return k


_k1 = _layer_kernel(W1, None)
_k2 = _layer_kernel(W2, C2)
_k3 = _final_kernel(W3, C3)


def _pack_edges(edge_index, edge_vals):
    pad = E_PAD - E
    dst = jnp.concatenate([edge_index[0], jnp.full((pad,), N, jnp.int32)])
    src = jnp.concatenate([edge_index[1], jnp.zeros((pad,), jnp.int32)])
    w_i = lax.bitcast_convert_type(
        jnp.concatenate([edge_vals, jnp.zeros((pad,), jnp.float32)]),
        jnp.int32)
    pk = jnp.stack([src, dst, w_i])
    pk = pk.reshape(3, NS, NBLK, EBLK).transpose(1, 2, 0, 3)
    return pk.reshape(NS * NBLK, 3 * EBLK)


@jax.jit
def kernel(user_emb, item_emb, edge_index, edge_vals):
    x0 = jnp.concatenate([user_emb, item_emb], axis=0)
    pk = _pack_edges(edge_index, edge_vals)
    x1 = _k1(pk, x0)
    x2 = _k2(pk, x1, x0)
    low, mid = _k3(pk, x0, x1, x2)
    out = jnp.concatenate([low, mid], axis=1)
    return out[:USER_NUM], out[USER_NUM:]


# speedup vs baseline: 1.2365x; 1.2365x over previous
import functools
import math

import jax
import jax.numpy as jnp
from jax import lax
from jax.experimental import pallas as pl
from jax.experimental.pallas import tpu as pltpu
from jax.experimental.pallas import tpu_sc as plsc

USER_NUM = 60000
ITEM_NUM = 40000
N = USER_NUM + ITEM_NUM
E = 1600000
D = 32

NC = 2
NS = 16
LANES = 16

HALF = N // NC
RPT = HALF // NS
DUMMY = HALF
ACC_ROWS = HALF + 8

CH = 128
EBLK = 2048
CPB = EBLK // CH
NBLK = 50
EPT = NBLK * EBLK
E_PAD = EPT * NS
NBUF = 2

RCH = 125
NRCH = RPT // RCH

_ALPHA = 3.0 * math.tanh(1.0 / 3.0)
W1 = 2.0 * _ALPHA
W2 = 1.875 * _ALPHA
C2 = -0.75 * _ALPHA * _ALPHA
W3 = (28.0 / 15.0) * _ALPHA
C3 = -0.8 * _ALPHA * _ALPHA
BETA = 2.0

_MESH = plsc.VectorSubcoreMesh(core_axis_name="c", subcore_axis_name="s")
_CPARAMS = pltpu.CompilerParams(use_tc_tiling_on_sc=False,
                                needs_layout_passes=False)

_EDGE_SCRATCH = [
    pltpu.VMEM_SHARED((ACC_ROWS, D), jnp.float32),
    pltpu.VMEM((3 * EBLK,), jnp.int32),
    pltpu.VMEM((NBUF, CH), jnp.int32),
    pltpu.VMEM((NBUF, CH), jnp.int32),
    pltpu.VMEM((NBUF, CH + LANES), jnp.float32),
    pltpu.VMEM((NBUF, CH, D), jnp.float32),
    pltpu.VMEM((RCH, D), jnp.float32),
    pltpu.VMEM((RCH, D), jnp.float32),
] + [pltpu.SemaphoreType.DMA] * NBUF


def _zero_acc(s, acc, zbuf):
    z16 = jnp.zeros((LANES,), jnp.float32)

    def fill(r, carry):
        zbuf[r, pl.ds(0, 16)] = z16
        zbuf[r, pl.ds(16, 16)] = z16
        return carry

    lax.fori_loop(0, RCH, fill, 0)
    base = s * RPT

    def body(k, carry):
        pltpu.sync_copy(zbuf, acc.at[pl.ds(base + k * RCH, RCH)])
        return carry

    lax.fori_loop(0, NRCH, body, 0)
    @pl.when(s == 0)
    def _():
        pltpu.sync_copy(zbuf.at[pl.ds(0, 8)], acc.at[pl.ds(HALF, 8)])


def _edge_phase(c, s, pk_hbm, x_hbm, acc,
                eb, sidx, lidx, wbuf, rows, gsems, wscale):
    half_lo = c * HALF

    def _prep_and_issue(ki, b):
        o = pl.multiple_of(ki * CH, 8)

        def prep(g2, cc):
            for j in range(2):
                go = (g2 * 2 + j) * LANES
                sl = pl.ds(o + go, LANES)
                sidx[b, pl.ds(go, LANES)] = eb[sl]
                dv = eb[pl.ds(EBLK + o + go, LANES)] - half_lo
                ok = (dv >= 0) & (dv < HALF)
                lidx[b, pl.ds(go, LANES)] = jnp.where(ok, dv, DUMMY)
                wv = plsc.bitcast(eb[pl.ds(2 * EBLK + o + go, LANES)],
                                  jnp.float32)
                wbuf[b, pl.ds(go, LANES)] = wv * wscale
            return cc

        lax.fori_loop(0, CH // LANES // 2, prep, 0)
        pltpu.async_copy(x_hbm.at[sidx.at[b]], rows.at[b], gsems[b])

    def _finish(b):
        pltpu.make_async_copy(x_hbm.at[pl.ds(0, CH)], rows.at[b],
                              gsems[b]).wait()

        def scale(q, cc):
            for j in range(4):
                e = q * 4 + j
                ws = wbuf[b, pl.ds(e, LANES)][0]
                rows[b, e, pl.ds(0, 16)] = rows[b, e, pl.ds(0, 16)] * ws
                rows[b, e, pl.ds(16, 16)] = rows[b, e, pl.ds(16, 16)] * ws
            return cc

        lax.fori_loop(0, CH // 4, scale, 0)
        pltpu.sync_copy(rows.at[b], acc.at[lidx.at[b]], add=True)

    def block(bi, carry):
        pltpu.sync_copy(pk_hbm.at[s * NBLK + bi], eb)
        _prep_and_issue(0, 0)

        def pair(kp, cc):
            ki0 = kp * 2

            @pl.when(kp > 0)
            def _():
                _finish(1)
            _prep_and_issue(ki0 + 1, 1)
            _finish(0)

            @pl.when(kp < CPB // 2 - 1)
            def _():
                _prep_and_issue(ki0 + 2, 0)
            return cc

        lax.fori_loop(0, CPB // 2, pair, 0)
        _finish(1)
        return carry

    lax.fori_loop(0, NBLK, block, 0)


def _layer_kernel(wscale, cprev):
    @functools.partial(
        pl.kernel,
        mesh=_MESH,
        out_type=jax.ShapeDtypeStruct((N, D), jnp.float32),
        scratch_types=_EDGE_SCRATCH,
        compiler_params=_CPARAMS,
    )
    def k(*refs):
        if cprev is None:
            pk_hbm, xc_hbm, out_hbm = refs[:3]
            xp_hbm = None
            rest = refs[3:]
        else:
            pk_hbm, xc_hbm, xp_hbm, out_hbm = refs[:4]
            rest = refs[4:]
        acc, eb, sidx, lidx, wbuf, rows, ta, tb = rest[:8]
        gsems = rest[8:]

        c = lax.axis_index("c")
        s = lax.axis_index("s")

        _zero_acc(s, acc, ta)
        plsc.subcore_barrier()
        _edge_phase(c, s, pk_hbm, xc_hbm, acc,
                    eb, sidx, lidx, wbuf, rows, gsems, wscale)
        plsc.subcore_barrier()

        lbase = s * RPT
        gbase = c * HALF + lbase

        def wb(k_, carry):
            lrow = lbase + k_ * RCH
            grow = gbase + k_ * RCH
            pltpu.sync_copy(acc.at[pl.ds(lrow, RCH)], ta)
            if cprev is not None:
                pltpu.sync_copy(xp_hbm.at[pl.ds(grow, RCH)], tb)

                def axpy(r, cc):
                    for h in (0, 16):
                        sl = pl.ds(h, 16)
                        ta[r, sl] = ta[r, sl] + tb[r, sl] * cprev
                    return cc

                lax.fori_loop(0, RCH, axpy, 0)
            pltpu.sync_copy(ta, out_hbm.at[pl.ds(grow, RCH)])
            return carry

        lax.fori_loop(0, NRCH, wb, 0)

    return k


def _final_kernel(wscale, cprev):
    @functools.partial(
        pl.kernel,
        mesh=_MESH,
        out_type=(jax.ShapeDtypeStruct((N, D), jnp.float32),
                  jax.ShapeDtypeStruct((N, D), jnp.float32)),
        scratch_types=_EDGE_SCRATCH,
        compiler_params=_CPARAMS,
    )
    def k(pk_hbm, x0_hbm, x1_hbm, x2_hbm, low_hbm, mid_hbm,
          acc, eb, sidx, lidx, wbuf, rows, ta, tb, *gsems):
        c = lax.axis_index("c")
        s = lax.axis_index("s")

        _zero_acc(s, acc, ta)
        plsc.subcore_barrier()
        _edge_phase(c, s, pk_hbm, x2_hbm, acc,
                    eb, sidx, lidx, wbuf, rows, gsems, wscale)
        plsc.subcore_barrier()

        lbase = s * RPT
        gbase = c * HALF + lbase
        c1 = 1.0 + cprev

        def wb(k_, carry):
            lrow = lbase + k_ * RCH
            grow = gbase + k_ * RCH
            pltpu.sync_copy(acc.at[pl.ds(lrow, RCH)], ta)
            pltpu.sync_copy(x1_hbm.at[pl.ds(grow, RCH)], tb)

            def add1(r, cc):
                for h in (0, 16):
                    sl = pl.ds(h, 16)
                    ta[r, sl] = ta[r, sl] + tb[r, sl] * c1
                return cc

            lax.fori_loop(0, RCH, add1, 0)
            pltpu.sync_copy(x2_hbm.at[pl.ds(grow, RCH)], tb)

            def add2(r, cc):
                for h in (0, 16):
                    sl = pl.ds(h, 16)
                    ta[r, sl] = ta[r, sl] + tb[r, sl]
                return cc

            lax.fori_loop(0, RCH, add2, 0)
            pltpu.sync_copy(x0_hbm.at[pl.ds(grow, RCH)], tb)

            def final(r, cc):
                for h in (0, 16):
                    sl = pl.ds(h, 16)
                    low = (ta[r, sl] + tb[r, sl]) * 0.25
                    ta[r, sl] = low
                    tb[r, sl] = tb[r, sl] * BETA - low
                return cc

            lax.fori_loop(0, RCH, final, 0)
            pltpu.sync_copy(ta, low_hbm.at[pl.ds(grow, RCH)])
            pltpu.sync_copy(tb, mid_hbm.at[pl.ds(grow, RCH)])
            return carry

        lax.fori_loop(0, NRCH, wb, 0)

    return k


_k1 = _layer_kernel(W1, None)
_k2 = _layer_kernel(W2, C2)
_k3 = _final_kernel(W3, C3)


def _pack_edges(edge_index, edge_vals):
    pad = E_PAD - E
    dst = jnp.concatenate([edge_index[0], jnp.full((pad,), N, jnp.int32)])
    src = jnp.concatenate([edge_index[1], jnp.zeros((pad,), jnp.int32)])
    w_i = lax.bitcast_convert_type(
        jnp.concatenate([edge_vals, jnp.zeros((pad,), jnp.float32)]),
        jnp.int32)
    pk = jnp.stack([src, dst, w_i])
    pk = pk.reshape(3, NS, NBLK, EBLK).transpose(1, 2, 0, 3)
    return pk.reshape(NS * NBLK, 3 * EBLK)


@jax.jit
def kernel(user_emb, item_emb, edge_index, edge_vals):
    x0 = jnp.concatenate([user_emb, item_emb], axis=0)
    pk = _pack_edges(edge_index, edge_vals)
    x1 = _k1(pk, x0)
    x2 = _k2(pk, x1, x0)
    low, mid = _k3(pk, x0, x1, x2)
    out = jnp.concatenate([low, mid], axis=1)
    return out[:USER_NUM], out[USER_NUM:]


# speedup vs baseline: 1.3442x; 1.0871x over previous
import functools
import math

import jax
import jax.numpy as jnp
from jax import lax
from jax.experimental import pallas as pl
from jax.experimental.pallas import tpu as pltpu
from jax.experimental.pallas import tpu_sc as plsc

USER_NUM = 60000
ITEM_NUM = 40000
N = USER_NUM + ITEM_NUM
E = 1600000
D = 32

NC = 2
NS = 16
LANES = 16

HALF = N // NC
RPT = HALF // NS
DUMMY = HALF
ACC_ROWS = HALF + 8

CH = 128
EBLK = 1920
CPB = EBLK // CH
NBLK = 53
EPT = NBLK * EBLK
E_PAD = EPT * NS
NBUF = 3

RCH = 125
NRCH = RPT // RCH

_ALPHA = 3.0 * math.tanh(1.0 / 3.0)
W1 = 2.0 * _ALPHA
W2 = 1.875 * _ALPHA
C2 = -0.75 * _ALPHA * _ALPHA
W3 = (28.0 / 15.0) * _ALPHA
C3 = -0.8 * _ALPHA * _ALPHA
BETA = 2.0

_MESH = plsc.VectorSubcoreMesh(core_axis_name="c", subcore_axis_name="s")
_CPARAMS = pltpu.CompilerParams(use_tc_tiling_on_sc=False,
                                needs_layout_passes=False)

_EDGE_SCRATCH = [
    pltpu.VMEM_SHARED((ACC_ROWS, D), jnp.float32),
    pltpu.VMEM((3 * EBLK,), jnp.int32),
    pltpu.VMEM((NBUF, CH), jnp.int32),
    pltpu.VMEM((NBUF, CH), jnp.int32),
    pltpu.VMEM((NBUF, CH + LANES), jnp.float32),
    pltpu.VMEM((NBUF, CH, D), jnp.float32),
    pltpu.VMEM((RCH, D), jnp.float32),
    pltpu.VMEM((RCH, D), jnp.float32),
] + [pltpu.SemaphoreType.DMA] * (2 * NBUF)


def _zero_acc(s, acc, zbuf):
    z16 = jnp.zeros((LANES,), jnp.float32)

    def fill(r, carry):
        zbuf[r, pl.ds(0, 16)] = z16
        zbuf[r, pl.ds(16, 16)] = z16
        return carry

    lax.fori_loop(0, RCH, fill, 0)
    base = s * RPT

    def body(k, carry):
        pltpu.sync_copy(zbuf, acc.at[pl.ds(base + k * RCH, RCH)])
        return carry

    lax.fori_loop(0, NRCH, body, 0)
    @pl.when(s == 0)
    def _():
        pltpu.sync_copy(zbuf.at[pl.ds(0, 8)], acc.at[pl.ds(HALF, 8)])


def _edge_phase(c, s, pk_hbm, x_hbm, acc,
                eb, sidx, lidx, wbuf, rows, sems, wscale):
    gsems = sems[:NBUF]
    ssems = sems[NBUF:]
    half_lo = c * HALF

    def _prep_and_issue(ki, b):
        o = pl.multiple_of(ki * CH, 8)

        def prep(g2, cc):
            for j in range(2):
                go = (g2 * 2 + j) * LANES
                sl = pl.ds(o + go, LANES)
                sidx[b, pl.ds(go, LANES)] = eb[sl]
                dv = eb[pl.ds(EBLK + o + go, LANES)] - half_lo
                ok = (dv >= 0) & (dv < HALF)
                lidx[b, pl.ds(go, LANES)] = jnp.where(ok, dv, DUMMY)
                wv = plsc.bitcast(eb[pl.ds(2 * EBLK + o + go, LANES)],
                                  jnp.float32)
                wbuf[b, pl.ds(go, LANES)] = wv * wscale
            return cc

        lax.fori_loop(0, CH // LANES // 2, prep, 0)
        pltpu.async_copy(x_hbm.at[sidx.at[b]], rows.at[b], gsems[b])

    def _finish(b):
        pltpu.make_async_copy(x_hbm.at[pl.ds(0, CH)], rows.at[b],
                              gsems[b]).wait()

        def scale(q, cc):
            for j in range(4):
                e = q * 4 + j
                ws = wbuf[b, pl.ds(e, LANES)][0]
                rows[b, e, pl.ds(0, 16)] = rows[b, e, pl.ds(0, 16)] * ws
                rows[b, e, pl.ds(16, 16)] = rows[b, e, pl.ds(16, 16)] * ws
            return cc

        lax.fori_loop(0, CH // 4, scale, 0)
        pltpu.async_copy(rows.at[b], acc.at[lidx.at[b]], ssems[b], add=True)

    def _drain_scatter(b):
        pltpu.make_async_copy(x_hbm.at[pl.ds(0, CH)], rows.at[b],
                              ssems[b]).wait()

    def block(bi, carry):
        pltpu.sync_copy(pk_hbm.at[s * NBLK + bi], eb)
        _prep_and_issue(0, 0)
        _prep_and_issue(1, 1)

        def ring(q, cc):
            for j in range(NBUF):
                ki = q * NBUF + j
                _finish(j)
                nxt = ki + NBUF - 1
                b2 = (j + NBUF - 1) % NBUF

                @pl.when(nxt >= NBUF)
                def _():
                    _drain_scatter(b2)

                @pl.when(nxt < CPB)
                def _():
                    _prep_and_issue(nxt, b2)
            return cc

        lax.fori_loop(0, CPB // NBUF, ring, 0)
        _drain_scatter((CPB - 1) % NBUF)
        return carry

    lax.fori_loop(0, NBLK, block, 0)


def _layer_kernel(wscale, cprev):
    @functools.partial(
        pl.kernel,
        mesh=_MESH,
        out_type=jax.ShapeDtypeStruct((N, D), jnp.float32),
        scratch_types=_EDGE_SCRATCH,
        compiler_params=_CPARAMS,
    )
    def k(*refs):
        if cprev is None:
            pk_hbm, xc_hbm, out_hbm = refs[:3]
            xp_hbm = None
            rest = refs[3:]
        else:
            pk_hbm, xc_hbm, xp_hbm, out_hbm = refs[:4]
            rest = refs[4:]
        acc, eb, sidx, lidx, wbuf, rows, ta, tb = rest[:8]
        gsems = rest[8:]

        c = lax.axis_index("c")
        s = lax.axis_index("s")

        _zero_acc(s, acc, ta)
        plsc.subcore_barrier()
        _edge_phase(c, s, pk_hbm, xc_hbm, acc,
                    eb, sidx, lidx, wbuf, rows, gsems, wscale)
        plsc.subcore_barrier()

        lbase = s * RPT
        gbase = c * HALF + lbase

        def wb(k_, carry):
            lrow = lbase + k_ * RCH
            grow = gbase + k_ * RCH
            pltpu.sync_copy(acc.at[pl.ds(lrow, RCH)], ta)
            if cprev is not None:
                pltpu.sync_copy(xp_hbm.at[pl.ds(grow, RCH)], tb)

                def axpy(r, cc):
                    for h in (0, 16):
                        sl = pl.ds(h, 16)
                        ta[r, sl] = ta[r, sl] + tb[r, sl] * cprev
                    return cc

                lax.fori_loop(0, RCH, axpy, 0)
            pltpu.sync_copy(ta, out_hbm.at[pl.ds(grow, RCH)])
            return carry

        lax.fori_loop(0, NRCH, wb, 0)

    return k


def _final_kernel(wscale, cprev):
    @functools.partial(
        pl.kernel,
        mesh=_MESH,
        out_type=(jax.ShapeDtypeStruct((N, D), jnp.float32),
                  jax.ShapeDtypeStruct((N, D), jnp.float32)),
        scratch_types=_EDGE_SCRATCH,
        compiler_params=_CPARAMS,
    )
    def k(pk_hbm, x0_hbm, x1_hbm, x2_hbm, low_hbm, mid_hbm,
          acc, eb, sidx, lidx, wbuf, rows, ta, tb, *gsems):
        c = lax.axis_index("c")
        s = lax.axis_index("s")

        _zero_acc(s, acc, ta)
        plsc.subcore_barrier()
        _edge_phase(c, s, pk_hbm, x2_hbm, acc,
                    eb, sidx, lidx, wbuf, rows, gsems, wscale)
        plsc.subcore_barrier()

        lbase = s * RPT
        gbase = c * HALF + lbase
        c1 = 1.0 + cprev

        def wb(k_, carry):
            lrow = lbase + k_ * RCH
            grow = gbase + k_ * RCH
            pltpu.sync_copy(acc.at[pl.ds(lrow, RCH)], ta)
            pltpu.sync_copy(x1_hbm.at[pl.ds(grow, RCH)], tb)

            def add1(r, cc):
                for h in (0, 16):
                    sl = pl.ds(h, 16)
                    ta[r, sl] = ta[r, sl] + tb[r, sl] * c1
                return cc

            lax.fori_loop(0, RCH, add1, 0)
            pltpu.sync_copy(x2_hbm.at[pl.ds(grow, RCH)], tb)

            def add2(r, cc):
                for h in (0, 16):
                    sl = pl.ds(h, 16)
                    ta[r, sl] = ta[r, sl] + tb[r, sl]
                return cc

            lax.fori_loop(0, RCH, add2, 0)
            pltpu.sync_copy(x0_hbm.at[pl.ds(grow, RCH)], tb)

            def final(r, cc):
                for h in (0, 16):
                    sl = pl.ds(h, 16)
                    low = (ta[r, sl] + tb[r, sl]) * 0.25
                    ta[r, sl] = low
                    tb[r, sl] = tb[r, sl] * BETA - low
                return cc

            lax.fori_loop(0, RCH, final, 0)
            pltpu.sync_copy(ta, low_hbm.at[pl.ds(grow, RCH)])
            pltpu.sync_copy(tb, mid_hbm.at[pl.ds(grow, RCH)])
            return carry

        lax.fori_loop(0, NRCH, wb, 0)

    return k


_k1 = _layer_kernel(W1, None)
_k2 = _layer_kernel(W2, C2)
_k3 = _final_kernel(W3, C3)


def _pack_edges(edge_index, edge_vals):
    pad = E_PAD - E
    dst = jnp.concatenate([edge_index[0], jnp.full((pad,), N, jnp.int32)])
    src = jnp.concatenate([edge_index[1], jnp.zeros((pad,), jnp.int32)])
    w_i = lax.bitcast_convert_type(
        jnp.concatenate([edge_vals, jnp.zeros((pad,), jnp.float32)]),
        jnp.int32)
    pk = jnp.stack([src, dst, w_i])
    pk = pk.reshape(3, NS, NBLK, EBLK).transpose(1, 2, 0, 3)
    return pk.reshape(NS * NBLK, 3 * EBLK)


@jax.jit
def kernel(user_emb, item_emb, edge_index, edge_vals):
    x0 = jnp.concatenate([user_emb, item_emb], axis=0)
    pk = _pack_edges(edge_index, edge_vals)
    x1 = _k1(pk, x0)
    x2 = _k2(pk, x1, x0)
    low, mid = _k3(pk, x0, x1, x2)
    out = jnp.concatenate([low, mid], axis=1)
    return out[:USER_NUM], out[USER_NUM:]
